# baseline (speedup 1.0000x reference)
import functools

import jax
import jax.numpy as jnp
from jax import lax
from jax.experimental import pallas as pl
from jax.experimental.pallas import tpu as pltpu
from jax.experimental.pallas import tpu_sc as plsc

NC = 2
NS = 16
NW = NC * NS

BATCH = 16384
HIST = 200
D = 64
B = BATCH * HIST
G = B // 128
GPW = G // NW
K = 4
ROWS = K * 128
CHUNKS = GPW // K


def _mesh():
    return plsc.VectorSubcoreMesh(
        core_axis_name="c", subcore_axis_name="s",
        num_cores=NC, num_subcores=NS)


@functools.partial(
    pl.kernel,
    out_type=jax.ShapeDtypeStruct((B, D), jnp.float32),
    mesh=_mesh(),
    scratch_types=[
        pltpu.VMEM((K, 128), jnp.int32),
        pltpu.VMEM((ROWS, D), jnp.float32),
        pltpu.SemaphoreType.DMA,
    ],
    compiler_params=pltpu.CompilerParams(use_tc_tiling_on_sc=False),
)
def _gather_kernel(idx_hbm, table_hbm, out_hbm, idx_v, rows_v, sem):
    wid = lax.axis_index("s") * NC + lax.axis_index("c")
    g0 = wid * GPW

    def body(i, _):
        g = g0 + i * K
        pltpu.sync_copy(idx_hbm.at[pl.ds(g, K)], idx_v)
        descs = [
            pltpu.async_copy(
                table_hbm.at[idx_v.at[j]],
                rows_v.at[pl.ds(j * 128, 128)], sem)
            for j in range(K)
        ]
        for d in descs:
            d.wait()
        pltpu.sync_copy(rows_v, out_hbm.at[pl.ds(g * 128, ROWS)])
        return ()

    lax.fori_loop(0, CHUNKS, body, ())


def kernel(tokens, embedding):
    idx = tokens.astype(jnp.int32).reshape(G, 128)
    out = _gather_kernel(idx, embedding)
    return out.reshape(BATCH, HIST, D)


# speedup vs baseline: 1.0712x; 1.0712x over previous
import functools

import jax
import jax.numpy as jnp
from jax import lax
from jax.experimental import pallas as pl
from jax.experimental.pallas import tpu as pltpu
from jax.experimental.pallas import tpu_sc as plsc

NC = 2
NS = 16
NW = NC * NS

BATCH = 16384
HIST = 200
D = 64
B = BATCH * HIST
G = B // 128
GPW = G // NW
K = 4
ROWS = K * 128
CHUNKS = GPW // K
NBUF = 2
STEPS = CHUNKS // NBUF


def _mesh():
    return plsc.VectorSubcoreMesh(
        core_axis_name="c", subcore_axis_name="s",
        num_cores=NC, num_subcores=NS)


@functools.partial(
    pl.kernel,
    out_type=jax.ShapeDtypeStruct((B, D), jnp.float32),
    mesh=_mesh(),
    scratch_types=[
        pltpu.VMEM((NBUF, K, 128), jnp.int32),
        pltpu.VMEM((NBUF, ROWS, D), jnp.float32),
        pltpu.SemaphoreType.DMA((NBUF,)),
        pltpu.SemaphoreType.DMA((NBUF,)),
        pltpu.SemaphoreType.DMA((NBUF,)),
    ],
    compiler_params=pltpu.CompilerParams(use_tc_tiling_on_sc=False),
)
def _gather_kernel(idx_hbm, table_hbm, out_hbm, idx_v, rows_v,
                   sem_i, sem_g, sem_o):
    wid = lax.axis_index("s") * NC + lax.axis_index("c")
    g0 = wid * GPW

    for b in range(NBUF):
        pltpu.async_copy(
            idx_hbm.at[pl.ds(g0 + b * K, K)], idx_v.at[b], sem_i.at[b])

    def body(i2, _):
        for b in range(NBUF):
            i = i2 * NBUF + b
            g = g0 + i * K

            @pl.when(i2 > 0)
            def _():
                pltpu.make_async_copy(
                    rows_v.at[b], out_hbm.at[pl.ds(g * 128, ROWS)],
                    sem_o.at[b]).wait()

            pltpu.make_async_copy(
                idx_hbm.at[pl.ds(g, K)], idx_v.at[b], sem_i.at[b]).wait()

            descs = [
                pltpu.async_copy(
                    table_hbm.at[idx_v.at[b, j]],
                    rows_v.at[b].at[pl.ds(j * 128, 128)], sem_g.at[b])
                for j in range(K)
            ]
            for d in descs:
                d.wait()

            @pl.when(i2 < STEPS - 1)
            def _():
                pltpu.async_copy(
                    idx_hbm.at[pl.ds(g + NBUF * K, K)], idx_v.at[b],
                    sem_i.at[b])

            pltpu.async_copy(
                rows_v.at[b], out_hbm.at[pl.ds(g * 128, ROWS)], sem_o.at[b])
        return ()

    lax.fori_loop(0, STEPS, body, ())

    for b in range(NBUF):
        pltpu.make_async_copy(
            rows_v.at[b], out_hbm.at[pl.ds(g0, ROWS)], sem_o.at[b]).wait()


def kernel(tokens, embedding):
    idx = tokens.astype(jnp.int32).reshape(G, 128)
    out = _gather_kernel(idx, embedding)
    return out.reshape(BATCH, HIST, D)


# speedup vs baseline: 1.0752x; 1.0038x over previous
import functools

import jax
import jax.numpy as jnp
from jax import lax
from jax.experimental import pallas as pl
from jax.experimental.pallas import tpu as pltpu
from jax.experimental.pallas import tpu_sc as plsc

NC = 2
NS = 16
NW = NC * NS

BATCH = 16384
HIST = 200
D = 64
B = BATCH * HIST
G = B // 128
GPW = G // NW
K = 4
ROWS = K * 128
CHUNKS = GPW // K
NBUF = 2
STEPS = CHUNKS // NBUF


def _mesh():
    return plsc.VectorSubcoreMesh(
        core_axis_name="c", subcore_axis_name="s",
        num_cores=NC, num_subcores=NS)


@functools.partial(
    pl.kernel,
    out_type=jax.ShapeDtypeStruct((B, D), jnp.float32),
    mesh=_mesh(),
    scratch_types=[
        pltpu.VMEM((NBUF, ROWS), jnp.int32),
        pltpu.VMEM((NBUF, ROWS, D), jnp.float32),
        pltpu.SemaphoreType.DMA((NBUF,)),
        pltpu.SemaphoreType.DMA((NBUF,)),
        pltpu.SemaphoreType.DMA((NBUF,)),
    ],
    compiler_params=pltpu.CompilerParams(use_tc_tiling_on_sc=False),
)
def _gather_kernel(idx_hbm, table_hbm, out_hbm, idx_v, rows_v,
                   sem_i, sem_g, sem_o):
    wid = lax.axis_index("s") * NC + lax.axis_index("c")
    g0 = wid * GPW

    for b in range(NBUF):
        pltpu.async_copy(
            idx_hbm.at[pl.ds((g0 + b * K) * 128, ROWS)], idx_v.at[b],
            sem_i.at[b])

    def body(i2, _):
        for b in range(NBUF):
            i = i2 * NBUF + b
            g = g0 + i * K

            @pl.when(i2 > 0)
            def _():
                pltpu.make_async_copy(
                    rows_v.at[b], out_hbm.at[pl.ds(g * 128, ROWS)],
                    sem_o.at[b]).wait()

            pltpu.make_async_copy(
                idx_hbm.at[pl.ds(g * 128, ROWS)], idx_v.at[b],
                sem_i.at[b]).wait()

            pltpu.async_copy(
                table_hbm.at[idx_v.at[b]], rows_v.at[b],
                sem_g.at[b]).wait()

            @pl.when(i2 < STEPS - 1)
            def _():
                pltpu.async_copy(
                    idx_hbm.at[pl.ds((g + NBUF * K) * 128, ROWS)],
                    idx_v.at[b], sem_i.at[b])

            pltpu.async_copy(
                rows_v.at[b], out_hbm.at[pl.ds(g * 128, ROWS)], sem_o.at[b])
        return ()

    lax.fori_loop(0, STEPS, body, ())

    for b in range(NBUF):
        pltpu.make_async_copy(
            rows_v.at[b], out_hbm.at[pl.ds(g0, ROWS)], sem_o.at[b]).wait()


def kernel(tokens, embedding):
    idx = tokens.astype(jnp.int32).reshape(B)
    out = _gather_kernel(idx, embedding)
    return out.reshape(BATCH, HIST, D)


# speedup vs baseline: 1.0764x; 1.0011x over previous
import functools

import jax
import jax.numpy as jnp
from jax import lax
from jax.experimental import pallas as pl
from jax.experimental.pallas import tpu as pltpu
from jax.experimental.pallas import tpu_sc as plsc

NC = 2
NS = 16
NW = NC * NS

BATCH = 16384
HIST = 200
D = 64
RPW = BATCH // NW
CR = 4
CHUNKS = RPW // CR
NBUF = 2
STEPS = CHUNKS // NBUF


def _mesh():
    return plsc.VectorSubcoreMesh(
        core_axis_name="c", subcore_axis_name="s",
        num_cores=NC, num_subcores=NS)


@functools.partial(
    pl.kernel,
    out_type=jax.ShapeDtypeStruct((BATCH, HIST, D), jnp.float32),
    mesh=_mesh(),
    scratch_types=[
        pltpu.VMEM((NBUF, CR, HIST), jnp.int32),
        pltpu.VMEM((NBUF, CR, HIST, D), jnp.float32),
        pltpu.SemaphoreType.DMA((NBUF,)),
        pltpu.SemaphoreType.DMA((NBUF,)),
        pltpu.SemaphoreType.DMA((NBUF,)),
    ],
    compiler_params=pltpu.CompilerParams(use_tc_tiling_on_sc=False),
)
def _gather_kernel(tok_hbm, table_hbm, out_hbm, idx_v, rows_v,
                   sem_i, sem_g, sem_o):
    wid = lax.axis_index("s") * NC + lax.axis_index("c")
    r0 = wid * RPW

    for b in range(NBUF):
        pltpu.async_copy(
            tok_hbm.at[pl.ds(r0 + b * CR, CR)], idx_v.at[b], sem_i.at[b])

    def body(i2, _):
        for b in range(NBUF):
            i = i2 * NBUF + b
            r = r0 + i * CR

            @pl.when(i2 > 0)
            def _():
                pltpu.make_async_copy(
                    rows_v.at[b], out_hbm.at[pl.ds(r, CR)],
                    sem_o.at[b]).wait()

            pltpu.make_async_copy(
                tok_hbm.at[pl.ds(r, CR)], idx_v.at[b], sem_i.at[b]).wait()

            descs = [
                pltpu.async_copy(
                    table_hbm.at[idx_v.at[b, j]], rows_v.at[b, j],
                    sem_g.at[b])
                for j in range(CR)
            ]
            for d in descs:
                d.wait()

            @pl.when(i2 < STEPS - 1)
            def _():
                pltpu.async_copy(
                    tok_hbm.at[pl.ds(r + NBUF * CR, CR)], idx_v.at[b],
                    sem_i.at[b])

            pltpu.async_copy(
                rows_v.at[b], out_hbm.at[pl.ds(r, CR)], sem_o.at[b])
        return ()

    lax.fori_loop(0, STEPS, body, ())

    for b in range(NBUF):
        pltpu.make_async_copy(
            rows_v.at[b], out_hbm.at[pl.ds(r0, CR)], sem_o.at[b]).wait()


def kernel(tokens, embedding):
    return _gather_kernel(tokens.astype(jnp.int32), embedding)
